# baseline (speedup 1.0000x reference)
import functools

import jax
import jax.numpy as jnp
from jax.experimental import pallas as pl
from jax.experimental.pallas import tpu as pltpu
from jax.experimental.pallas import tpu_sc as plsc

N = 5000
NPAD = 5120
B = 256
NB = NPAD // B
THRESHOLD = 0.5
MAX_SIZE = 300


OUTW = 384


def _nms_step(bxs_ref, order_ref, out_ref, crd_ref, keep_ref, acc_ref,
              run_ref):
    k = pl.program_id(0)

    @pl.when(k == 0)
    def _init():
        keep_ref[...] = jnp.zeros_like(keep_ref)
        acc_ref[...] = jnp.zeros_like(acc_ref)
        run_ref[0, 0] = 0.0

    base = k * B
    cx1 = bxs_ref[pl.ds(base, B), 0:1]
    cy1 = bxs_ref[pl.ds(base, B), 1:2]
    cx2 = bxs_ref[pl.ds(base, B), 2:3]
    cy2 = bxs_ref[pl.ds(base, B), 3:4]
    careas = (cx2 - cx1) * (cy2 - cy1)
    rx1 = jnp.reshape(cx1, (1, B))
    ry1 = jnp.reshape(cy1, (1, B))
    rx2 = jnp.reshape(cx2, (1, B))
    ry2 = jnp.reshape(cy2, (1, B))
    rareas = (rx2 - rx1) * (ry2 - ry1)
    crd_ref[0:1, pl.ds(base, B)] = rx1
    crd_ref[1:2, pl.ds(base, B)] = ry1
    crd_ref[2:3, pl.ds(base, B)] = rx2
    crd_ref[3:4, pl.ds(base, B)] = ry2

    def pbody(jb, acc):
        jbase = jb * B
        jx1 = crd_ref[0:1, pl.ds(jbase, B)]
        jy1 = crd_ref[1:2, pl.ds(jbase, B)]
        jx2 = crd_ref[2:3, pl.ds(jbase, B)]
        jy2 = crd_ref[3:4, pl.ds(jbase, B)]
        jareas = (jx2 - jx1) * (jy2 - jy1)
        w = jnp.maximum(jnp.minimum(cx2, jx2) - jnp.maximum(cx1, jx1), 0.0)
        h = jnp.maximum(jnp.minimum(cy2, jy2) - jnp.maximum(cy1, jy1), 0.0)
        inter = w * h
        iou = inter / (careas + jareas - inter + 1e-9)
        kr = keep_ref[0:1, pl.ds(jbase, B)]
        return acc + jnp.where(iou > THRESHOLD, 1.0, 0.0) * kr

    acc = jax.lax.fori_loop(0, k, pbody, jnp.zeros((B, B), jnp.float32))
    supp = jnp.sum(acc, axis=1, keepdims=True)
    keep_cur = jnp.reshape((supp == 0.0).astype(jnp.float32), (1, B))

    bw = jnp.maximum(jnp.minimum(cx2, rx2) - jnp.maximum(cx1, rx1), 0.0)
    bh = jnp.maximum(jnp.minimum(cy2, ry2) - jnp.maximum(cy1, ry1), 0.0)
    binter = bw * bh
    biou = binter / (careas + rareas - binter + 1e-9)
    rowi = jax.lax.broadcasted_iota(jnp.int32, (B, B), 0)
    coli = jax.lax.broadcasted_iota(jnp.int32, (B, B), 1)
    ts = jnp.where((biou > THRESHOLD) & (rowi < coli), 1.0, 0.0)
    l0 = jnp.zeros_like(keep_cur)

    def fcond(carry):
        it, s = carry
        return jnp.logical_and(
            it < B,
            jnp.sum((s[0:1, :] != s[1:2, :]).astype(jnp.float32)) > 0.0)

    def fbody(carry):
        it, s = carry
        r = jnp.dot(s, ts, preferred_element_type=jnp.float32)
        lnew = keep_cur * (r[1:2, :] == 0.0).astype(jnp.float32)
        unew = keep_cur * (r[0:1, :] == 0.0).astype(jnp.float32)
        return it + 1, jnp.concatenate([lnew, unew], axis=0)

    _, s = jax.lax.while_loop(
        fcond, fbody, (0, jnp.concatenate([l0, keep_cur], axis=0)))
    keep_cur = s[0:1, :]

    keep_ref[0:1, pl.ds(base, B)] = keep_cur

    lanei = jax.lax.broadcasted_iota(jnp.int32, (1, B), 1)
    validf = jnp.where(base + lanei < N, 1.0, 0.0)
    kv = keep_cur * validf
    ltm = jnp.where(rowi < coli, 1.0, 0.0)
    run = run_ref[0, 0]
    grank = jnp.dot(kv, ltm, preferred_element_type=jnp.float32) + run
    grankt = jnp.reshape(grank, (B, 1))
    kvt = jnp.reshape(kv, (B, 1))
    oiota = jax.lax.broadcasted_iota(jnp.int32, (1, OUTW), 1).astype(
        jnp.float32)
    oneh = jnp.where(grankt == oiota, 1.0, 0.0) * kvt
    ordc = order_ref[0:1, pl.ds(base, B)]
    acc_ref[...] += jnp.dot(ordc, oneh, preferred_element_type=jnp.float32,
                            precision=jax.lax.Precision.HIGHEST)
    run_ref[0, 0] = run + jnp.sum(kv)

    @pl.when(k == NB - 1)
    def _emit():
        total = run_ref[0, 0]
        o0 = order_ref[0, 0]
        out_ref[...] = acc_ref[...] + jnp.where(oiota >= total, o0, 0.0)


@functools.partial(jax.jit, static_argnames=())
def _nms_kept(bxs, orderf):
    return pl.pallas_call(
        _nms_step,
        grid=(NB,),
        in_specs=[
            pl.BlockSpec((NPAD, 16), lambda k: (0, 0)),
            pl.BlockSpec((1, NPAD), lambda k: (0, 0)),
        ],
        out_specs=pl.BlockSpec((1, OUTW), lambda k: (0, 0)),
        out_shape=jax.ShapeDtypeStruct((1, OUTW), jnp.float32),
        scratch_shapes=[
            pltpu.VMEM((4, NPAD), jnp.float32),
            pltpu.VMEM((1, NPAD), jnp.float32),
            pltpu.VMEM((1, OUTW), jnp.float32),
            pltpu.SMEM((1, 1), jnp.float32),
        ],
    )(bxs, orderf)


_NW = 32
_CHUNK = NPAD // _NW
_L = 16


def _sc_gather_body(rois16_hbm, order_hbm, bxs_hbm, ordf_hbm,
                    idx_v, rows_v, ordf_v, sem):
    wid = jax.lax.axis_index("s") * 2 + jax.lax.axis_index("c")
    base = wid * _CHUNK
    pltpu.sync_copy(order_hbm.at[pl.ds(base, _CHUNK)], idx_v)
    pltpu.async_copy(rois16_hbm.at[idx_v], rows_v, sem).wait()
    for o in range(0, _CHUNK, _L):
        ordf_v[pl.ds(o, _L)] = idx_v[pl.ds(o, _L)].astype(jnp.float32)
    pltpu.sync_copy(rows_v, bxs_hbm.at[pl.ds(base, _CHUNK)])
    pltpu.sync_copy(ordf_v, ordf_hbm.at[pl.ds(base, _CHUNK)])


@jax.jit
def _sc_gather(rois16, order_pad):
    return pl.kernel(
        _sc_gather_body,
        mesh=plsc.VectorSubcoreMesh(core_axis_name="c", subcore_axis_name="s"),
        compiler_params=pltpu.CompilerParams(use_tc_tiling_on_sc=False),
        out_type=(
            jax.ShapeDtypeStruct((NPAD, 16), jnp.float32),
            jax.ShapeDtypeStruct((NPAD,), jnp.float32),
        ),
        scratch_types=[
            pltpu.VMEM((_CHUNK,), jnp.int32),
            pltpu.VMEM((_CHUNK, 16), jnp.float32),
            pltpu.VMEM((_CHUNK,), jnp.float32),
            pltpu.SemaphoreType.DMA,
        ],
    )(rois16, order_pad)


def kernel(rois, scores):
    order = jnp.argsort(-scores)
    order_pad = jnp.concatenate(
        [order, jnp.zeros((NPAD - N,), order.dtype)])
    rois16 = jnp.pad(rois, ((0, 0), (0, 12)))
    bxs, orderf = _sc_gather(rois16, order_pad)
    out = _nms_kept(bxs, orderf.reshape(1, NPAD))
    return out[0, :MAX_SIZE].astype(order.dtype)


# speedup vs baseline: 1.1582x; 1.1582x over previous
import functools

import jax
import jax.numpy as jnp
from jax.experimental import pallas as pl
from jax.experimental.pallas import tpu as pltpu
from jax.experimental.pallas import tpu_sc as plsc

N = 5000
NPAD = 5120
B = 256
NB = NPAD // B
THRESHOLD = 0.5
MAX_SIZE = 300


OUTW = 384


def _nms_step(bxs_ref, order_ref, out_ref, crd_ref, keep_ref, acc_ref,
              run_ref):
    k = pl.program_id(0)

    @pl.when(k == 0)
    def _init():
        keep_ref[...] = jnp.zeros_like(keep_ref)
        acc_ref[...] = jnp.zeros_like(acc_ref)
        run_ref[0, 0] = 0.0
        for c in range(4):
            for jb in range(NB):
                crd_ref[c, pl.ds(jb * B, B)] = jnp.reshape(
                    bxs_ref[pl.ds(jb * B, B), c], (B,))

    base = k * B
    rx1 = crd_ref[0:1, pl.ds(base, B)]
    ry1 = crd_ref[1:2, pl.ds(base, B)]
    rx2 = crd_ref[2:3, pl.ds(base, B)]
    ry2 = crd_ref[3:4, pl.ds(base, B)]
    rareas = (rx2 - rx1) * (ry2 - ry1)
    cx1 = jnp.reshape(rx1, (B, 1))
    cy1 = jnp.reshape(ry1, (B, 1))
    cx2 = jnp.reshape(rx2, (B, 1))
    cy2 = jnp.reshape(ry2, (B, 1))
    careas = (cx2 - cx1) * (cy2 - cy1)

    def pbody(jb, acc):
        jbase = jb * B
        jx1 = crd_ref[0:1, pl.ds(jbase, B)]
        jy1 = crd_ref[1:2, pl.ds(jbase, B)]
        jx2 = crd_ref[2:3, pl.ds(jbase, B)]
        jy2 = crd_ref[3:4, pl.ds(jbase, B)]
        jareas = (jx2 - jx1) * (jy2 - jy1)
        w = jnp.maximum(jnp.minimum(cx2, jx2) - jnp.maximum(cx1, jx1), 0.0)
        h = jnp.maximum(jnp.minimum(cy2, jy2) - jnp.maximum(cy1, jy1), 0.0)
        inter = w * h
        iou = inter / (careas + jareas - inter + 1e-9)
        kr = keep_ref[0:1, pl.ds(jbase, B)]
        return acc + jnp.where(iou > THRESHOLD, 1.0, 0.0) * kr

    acc = jax.lax.fori_loop(0, k, pbody, jnp.zeros((B, B), jnp.float32))
    supp = jnp.sum(acc, axis=1, keepdims=True)
    keep_cur = jnp.reshape((supp == 0.0).astype(jnp.float32), (1, B))

    bw = jnp.maximum(jnp.minimum(cx2, rx2) - jnp.maximum(cx1, rx1), 0.0)
    bh = jnp.maximum(jnp.minimum(cy2, ry2) - jnp.maximum(cy1, ry1), 0.0)
    binter = bw * bh
    biou = binter / (careas + rareas - binter + 1e-9)
    rowi = jax.lax.broadcasted_iota(jnp.int32, (B, B), 0)
    coli = jax.lax.broadcasted_iota(jnp.int32, (B, B), 1)
    ts = jnp.where((biou > THRESHOLD) & (rowi < coli), 1.0, 0.0)
    l0 = jnp.zeros_like(keep_cur)

    def fcond(carry):
        it, s = carry
        return jnp.logical_and(
            it < B,
            jnp.sum((s[0:1, :] != s[1:2, :]).astype(jnp.float32)) > 0.0)

    def fbody(carry):
        it, s = carry
        r = jnp.dot(s, ts, preferred_element_type=jnp.float32)
        lnew = keep_cur * (r[1:2, :] == 0.0).astype(jnp.float32)
        unew = keep_cur * (r[0:1, :] == 0.0).astype(jnp.float32)
        return it + 1, jnp.concatenate([lnew, unew], axis=0)

    _, s = jax.lax.while_loop(
        fcond, fbody, (0, jnp.concatenate([l0, keep_cur], axis=0)))
    keep_cur = s[0:1, :]

    keep_ref[0:1, pl.ds(base, B)] = keep_cur

    lanei = jax.lax.broadcasted_iota(jnp.int32, (1, B), 1)
    validf = jnp.where(base + lanei < N, 1.0, 0.0)
    kv = keep_cur * validf
    ltm = jnp.where(rowi < coli, 1.0, 0.0)
    run = run_ref[0, 0]
    grank = jnp.dot(kv, ltm, preferred_element_type=jnp.float32) + run
    grankt = jnp.reshape(grank, (B, 1))
    kvt = jnp.reshape(kv, (B, 1))
    oiota = jax.lax.broadcasted_iota(jnp.int32, (1, OUTW), 1).astype(
        jnp.float32)
    oneh = jnp.where(grankt == oiota, 1.0, 0.0) * kvt
    ordc = order_ref[0:1, pl.ds(base, B)]
    acc_ref[...] += jnp.dot(ordc, oneh, preferred_element_type=jnp.float32,
                            precision=jax.lax.Precision.HIGHEST)
    run_ref[0, 0] = run + jnp.sum(kv)

    @pl.when(k == NB - 1)
    def _emit():
        total = run_ref[0, 0]
        o0 = order_ref[0, 0]
        out_ref[...] = acc_ref[...] + jnp.where(oiota >= total, o0, 0.0)


@functools.partial(jax.jit, static_argnames=())
def _nms_kept(bxs, orderf):
    return pl.pallas_call(
        _nms_step,
        grid=(NB,),
        in_specs=[
            pl.BlockSpec((NPAD, 4), lambda k: (0, 0)),
            pl.BlockSpec((1, NPAD), lambda k: (0, 0)),
        ],
        out_specs=pl.BlockSpec((1, OUTW), lambda k: (0, 0)),
        out_shape=jax.ShapeDtypeStruct((1, OUTW), jnp.float32),
        scratch_shapes=[
            pltpu.VMEM((4, NPAD), jnp.float32),
            pltpu.VMEM((1, NPAD), jnp.float32),
            pltpu.VMEM((1, OUTW), jnp.float32),
            pltpu.SMEM((1, 1), jnp.float32),
        ],
    )(bxs, orderf)


_NW = 32
_CHUNK = NPAD // _NW
_L = 16


def _sc_gather_body(rois16_hbm, order_hbm, bxs_hbm, ordf_hbm,
                    idx_v, rows_v, ordf_v, sem):
    wid = jax.lax.axis_index("s") * 2 + jax.lax.axis_index("c")
    base = wid * _CHUNK
    pltpu.sync_copy(order_hbm.at[pl.ds(base, _CHUNK)], idx_v)
    pltpu.async_copy(rois16_hbm.at[idx_v], rows_v, sem).wait()
    for o in range(0, _CHUNK, _L):
        ordf_v[pl.ds(o, _L)] = idx_v[pl.ds(o, _L)].astype(jnp.float32)
    pltpu.sync_copy(rows_v, bxs_hbm.at[pl.ds(base, _CHUNK)])
    pltpu.sync_copy(ordf_v, ordf_hbm.at[pl.ds(base, _CHUNK)])


@jax.jit
def _sc_gather(rois, order_pad):
    return pl.kernel(
        _sc_gather_body,
        mesh=plsc.VectorSubcoreMesh(core_axis_name="c", subcore_axis_name="s"),
        compiler_params=pltpu.CompilerParams(use_tc_tiling_on_sc=False),
        out_type=(
            jax.ShapeDtypeStruct((NPAD, 4), jnp.float32),
            jax.ShapeDtypeStruct((NPAD,), jnp.float32),
        ),
        scratch_types=[
            pltpu.VMEM((_CHUNK,), jnp.int32),
            pltpu.VMEM((_CHUNK, 4), jnp.float32),
            pltpu.VMEM((_CHUNK,), jnp.float32),
            pltpu.SemaphoreType.DMA,
        ],
    )(rois, order_pad)


def kernel(rois, scores):
    order = jnp.argsort(-scores)
    order_pad = jnp.concatenate(
        [order, jnp.zeros((NPAD - N,), order.dtype)])
    bxs, orderf = _sc_gather(rois, order_pad)
    out = _nms_kept(bxs, orderf.reshape(1, NPAD))
    return out[0, :MAX_SIZE].astype(order.dtype)
